# baseline (speedup 1.0000x reference)
import functools

import jax
import jax.numpy as jnp
from jax import lax
from jax.experimental import pallas as pl
from jax.experimental.pallas import tpu as pltpu
from jax.experimental.pallas import tpu_sc as plsc

NWK = 16
NW = 32
CW = 128
NBUF = 4
LEAD = 3


def _sc_gather_transposed(idx3, table):
    nwk, nch, cw = idx3.shape
    v, d = table.shape
    nh = nch // 2
    ndt = d // 8
    nsteps = nh // NBUF
    assert nh % NBUF == 0 and nsteps >= 2 and LEAD < NBUF and cw == CW

    mesh = plsc.VectorSubcoreMesh(
        core_axis_name="c", subcore_axis_name="s", num_cores=1)

    @functools.partial(
        pl.kernel,
        mesh=mesh,
        compiler_params=pltpu.CompilerParams(
            use_tc_tiling_on_sc=False, needs_layout_passes=False),
        out_type=jax.ShapeDtypeStruct((nh, ndt, NW, 8 * cw), jnp.float32),
        scratch_types=(
            [pltpu.VMEM((nch, cw), jnp.int32),
             pltpu.VMEM((NBUF, cw, d), jnp.float32),
             pltpu.VMEM((NBUF, ndt, 8 * cw), jnp.float32)]
            + [pltpu.SemaphoreType.DMA] * (2 * NBUF)
        ),
    )
    def k(idx_hbm, table_hbm, out_hbm, idx_v, rows_v, tr_v, *sems):
        gsem = sems[:NBUF]
        osem = sems[NBUF:]
        wid = lax.axis_index("s")
        pltpu.sync_copy(idx_hbm.at[wid], idx_v)

        lane = lax.iota(jnp.int32, 16)
        bidx = [lane + g * 16 for g in range(cw // 16)]

        def transpose(b):
            rows = rows_v.at[b]
            tr = tr_v.at[b]

            @plsc.parallel_loop(0, d, 1, unroll=8)
            def tbody(dd):
                rot = (lane + dd) & (d - 1)
                rot_q = rot >> 3
                rot_low = (rot & 7) * cw
                for g in range(cw // 16):
                    vals = plsc.load_gather(rows, [bidx[g], rot])
                    plsc.store_scatter(tr, [rot_q, rot_low + bidx[g]], vals)

        def pipeline(off, blk):
            def start_gather(j, b):
                pltpu.async_copy(
                    table_hbm.at[idx_v.at[off + j]], rows_v.at[b], gsem[b])

            def wait_gather(j, b):
                pltpu.make_async_copy(
                    table_hbm.at[idx_v.at[off + j]], rows_v.at[b],
                    gsem[b]).wait()

            def start_ocopy(j, b):
                pltpu.async_copy(tr_v.at[b], out_hbm.at[j, :, blk], osem[b])

            def wait_ocopy(j, b):
                pltpu.make_async_copy(
                    tr_v.at[b], out_hbm.at[j, :, blk], osem[b]).wait()

            for bb in range(LEAD):
                start_gather(bb, bb)

            def round_body(s, carry):
                for bb in range(NBUF):
                    j = s * NBUF + bb

                    wait_gather(j, bb)

                    @pl.when(j + LEAD < nh)
                    def _():
                        start_gather(j + LEAD, (bb + LEAD) % NBUF)

                    @pl.when(j >= NBUF)
                    def _():
                        wait_ocopy(j - NBUF, bb)

                    transpose(bb)
                    start_ocopy(j, bb)
                return carry

            lax.fori_loop(0, nsteps, round_body, 0)

            for bb in range(NBUF):
                wait_ocopy((nsteps - 1) * NBUF + bb, bb)

        pipeline(0, wid)
        pipeline(nh, wid + NWK)

    return k(idx3, table)


def kernel(x, embedding):
    bsz, hist = x.shape
    d = embedding.shape[1]
    idx3 = (x.T.astype(jnp.int32)
            .reshape(hist, 2, NWK, bsz // NW)
            .transpose(2, 1, 0, 3)
            .reshape(NWK, 2 * hist, bsz // NW))
    out4 = _sc_gather_transposed(idx3, embedding)
    out5 = out4.reshape(hist, d // 8, NW, 8, bsz // NW)
    return out5.transpose(2, 4, 0, 1, 3).reshape(bsz, hist, d)


# speedup vs baseline: 1.1342x; 1.1342x over previous
import functools

import jax
import jax.numpy as jnp
from jax import lax
from jax.experimental import pallas as pl
from jax.experimental.pallas import tpu as pltpu
from jax.experimental.pallas import tpu_sc as plsc

NC = 2
NS = 16
NW = NC * NS
CW = 128
NBUF = 5
LEAD = 4


def _sc_gather_transposed(idx3, table):
    nw, nh, cw = idx3.shape
    v, d = table.shape
    ndt = d // 8
    nsteps = nh // NBUF
    assert nh % NBUF == 0 and nsteps >= 2 and LEAD < NBUF and cw == CW

    mesh = plsc.VectorSubcoreMesh(core_axis_name="c", subcore_axis_name="s")

    @functools.partial(
        pl.kernel,
        mesh=mesh,
        compiler_params=pltpu.CompilerParams(
            use_tc_tiling_on_sc=False, needs_layout_passes=False),
        out_type=jax.ShapeDtypeStruct((nh, ndt, nw, 8 * cw), jnp.float32),
        scratch_types=(
            [pltpu.VMEM((nh, cw), jnp.int32),
             pltpu.VMEM((NBUF, cw, d), jnp.float32),
             pltpu.VMEM((NBUF, ndt, 8 * cw), jnp.float32)]
            + [pltpu.SemaphoreType.DMA] * (2 * NBUF)
        ),
    )
    def k(idx_hbm, table_hbm, out_hbm, idx_v, rows_v, tr_v, *sems):
        gsem = sems[:NBUF]
        osem = sems[NBUF:]
        wid = lax.axis_index("s") * NC + lax.axis_index("c")
        pltpu.sync_copy(idx_hbm.at[wid], idx_v)

        lane = lax.iota(jnp.int32, 16)
        bidx = [lane + g * 16 for g in range(cw // 16)]

        def start_gather(j, b):
            pltpu.async_copy(table_hbm.at[idx_v.at[j]], rows_v.at[b], gsem[b])

        def wait_gather(j, b):
            pltpu.make_async_copy(
                table_hbm.at[idx_v.at[j]], rows_v.at[b], gsem[b]).wait()

        def out_slice(j):
            return out_hbm.at[j, :, wid]

        def start_ocopy(j, b):
            pltpu.async_copy(tr_v.at[b], out_slice(j), osem[b])

        def wait_ocopy(j, b):
            pltpu.make_async_copy(tr_v.at[b], out_slice(j), osem[b]).wait()

        def transpose(b):
            rows = rows_v.at[b]
            tr = tr_v.at[b]

            @plsc.parallel_loop(0, d, 1, unroll=8)
            def tbody(dd):
                rot = (lane + dd) & (d - 1)
                rot_q = rot >> 3
                rot_low = (rot & 7) * cw
                for g in range(cw // 16):
                    vals = plsc.load_gather(rows, [bidx[g], rot])
                    plsc.store_scatter(tr, [rot_q, rot_low + bidx[g]], vals)

        for bb in range(LEAD):
            start_gather(bb, bb)

        def round_body(s, carry):
            for bb in range(NBUF):
                j = s * NBUF + bb

                wait_gather(j, bb)

                @pl.when(j + LEAD < nh)
                def _():
                    start_gather(j + LEAD, (bb + LEAD) % NBUF)

                @pl.when(j >= NBUF)
                def _():
                    wait_ocopy(j - NBUF, bb)

                transpose(bb)
                start_ocopy(j, bb)
            return carry

        lax.fori_loop(0, nsteps, round_body, 0)

        for bb in range(NBUF):
            wait_ocopy((nsteps - 1) * NBUF + bb, bb)

    return k(idx3, table)


def kernel(x, embedding):
    bsz, hist = x.shape
    d = embedding.shape[1]
    idx3 = x.T.astype(jnp.int32).reshape(hist, NW, bsz // NW).transpose(1, 0, 2)
    out4 = _sc_gather_transposed(idx3, embedding)
    out5 = out4.reshape(hist, d // 8, NW, 8, bsz // NW)
    return out5.transpose(2, 4, 0, 1, 3).reshape(bsz, hist, d)
